# baseline (speedup 1.0000x reference)
import functools

import jax
import jax.numpy as jnp
from jax import lax
from jax.experimental import pallas as pl
from jax.experimental.pallas import tpu as pltpu
from jax.experimental.pallas import tpu_sc as plsc

_N = 10000
_E = 320000
_D = 128
_DO = 40
_DOP = 128

_NC = 2
_NS = 16
_NW = _NC * _NS
_EPT = _E // _NW
_CH = 40
_NCHUNK = _EPT // _CH
_NBUF = 5
_K = 3
_ZR = 8
_CHD = 80
_NCHD = _EPT // _CHD
_RA = 624
_TAILB = _RA * _NS

_mesh = plsc.VectorSubcoreMesh(core_axis_name="c", subcore_axis_name="s")
_sc_params = pltpu.CompilerParams(needs_layout_passes=False)


def _zero_rows(zbuf, acc, s, feat):
    @pl.loop(0, _ZR)
    def _(i):
        @pl.loop(0, feat // 16)
        def _(j):
            zbuf[i, pl.ds(j * 16, 16)] = jnp.zeros((16,), jnp.float32)

    rbase = s * _RA

    @pl.loop(0, _RA // _ZR)
    def _(k):
        pltpu.sync_copy(zbuf, acc.at[pl.ds(rbase + k * _ZR, _ZR)])

    @pl.when(s == _NS - 1)
    def _():
        @pl.loop(0, (_N - _TAILB) // _ZR)
        def _(k):
            pltpu.sync_copy(zbuf, acc.at[pl.ds(_TAILB + k * _ZR, _ZR)])


def _copy_out(acc, out_hbm, c, s):
    rbase = s * _RA
    pltpu.sync_copy(acc.at[pl.ds(rbase, _RA)],
                    out_hbm.at[pl.ds(c * _N + rbase, _RA)])

    @pl.when(s == _NS - 1)
    def _():
        pltpu.sync_copy(acc.at[pl.ds(_TAILB, _N - _TAILB)],
                        out_hbm.at[pl.ds(c * _N + _TAILB, _N - _TAILB)])


def _make_sc_scatter(feat):
    @functools.partial(
        pl.kernel,
        out_type=jax.ShapeDtypeStruct((_NC * _N, feat), jnp.float32),
        mesh=_mesh,
        scratch_types=[
            pltpu.VMEM((_EPT,), jnp.int32),
            pltpu.VMEM((_EPT,), jnp.int32),
            pltpu.VMEM((_NBUF, _CH, feat), jnp.float32),
            pltpu.VMEM((_ZR, feat), jnp.float32),
            pltpu.VMEM_SHARED((_N, feat), jnp.float32),
            pltpu.SemaphoreType.DMA((_NBUF,)),
            pltpu.SemaphoreType.DMA((_NBUF,)),
        ],
        compiler_params=_sc_params,
    )
    def sc_scatter(u_hbm, src_hbm, dst_hbm, out_hbm, sidx, didx, rows, zbuf,
                   acc, gsem, ssem):
        c = lax.axis_index("c")
        s = lax.axis_index("s")
        wid = c * _NS + s
        pltpu.sync_copy(src_hbm.at[pl.ds(wid * _EPT, _EPT)], sidx)
        pltpu.sync_copy(dst_hbm.at[pl.ds(wid * _EPT, _EPT)], didx)
        _zero_rows(zbuf, acc, s, feat)
        plsc.subcore_barrier()

        def gather(g, b):
            pltpu.async_copy(u_hbm.at[sidx.at[pl.ds(g * _CH, _CH)]],
                             rows.at[b], gsem.at[b])

        def gather_wait(g, b):
            pltpu.make_async_copy(u_hbm.at[sidx.at[pl.ds(g * _CH, _CH)]],
                                  rows.at[b], gsem.at[b]).wait()

        def scatter(g, b):
            pltpu.async_copy(rows.at[b], acc.at[didx.at[pl.ds(g * _CH, _CH)]],
                             ssem.at[b], add=True)

        def scatter_wait(g, b):
            pltpu.make_async_copy(rows.at[b],
                                  acc.at[didx.at[pl.ds(g * _CH, _CH)]],
                                  ssem.at[b]).wait()

        for b in range(_K):
            gather(b, b)

        @pl.loop(0, _NCHUNK // _NBUF)
        def _(r):
            g0 = r * _NBUF
            for b in range(_NBUF):
                g = g0 + b
                b2 = (b + _K) % _NBUF

                @pl.when(g + _K < _NCHUNK)
                def _():
                    @pl.when(g >= _NBUF - _K)
                    def _():
                        scatter_wait(g - (_NBUF - _K), b2)
                    gather(g + _K, b2)

                gather_wait(g, b)
                scatter(g, b)

        for b in range(_NBUF):
            scatter_wait(_NCHUNK - _NBUF + b, b)

        plsc.subcore_barrier()
        _copy_out(acc, out_hbm, c, s)

    return sc_scatter


_sc_scatter_d = _make_sc_scatter(_D)
_sc_scatter_p = _sc_scatter_d


@functools.partial(
    pl.kernel,
    out_type=jax.ShapeDtypeStruct((_NC * _N, 16), jnp.float32),
    mesh=_mesh,
    scratch_types=[
        pltpu.VMEM((_EPT,), jnp.int32),
        pltpu.VMEM((_CHD, 16), jnp.float32),
        pltpu.VMEM((_ZR, 16), jnp.float32),
        pltpu.VMEM_SHARED((_N, 16), jnp.float32),
        pltpu.SemaphoreType.DMA,
    ],
    compiler_params=_sc_params,
)
def _sc_deg(dst_hbm, out_hbm, didx, vbuf, zbuf, acc, dsem):
    c = lax.axis_index("c")
    s = lax.axis_index("s")
    wid = c * _NS + s
    e0 = (lax.iota(jnp.int32, 16) == 0).astype(jnp.float32)

    @pl.loop(0, _CHD)
    def _(i):
        vbuf[i, pl.ds(0, 16)] = e0

    pltpu.sync_copy(dst_hbm.at[pl.ds(wid * _EPT, _EPT)], didx)
    _zero_rows(zbuf, acc, s, 16)
    plsc.subcore_barrier()

    @pl.loop(0, _NCHD // _NBUF)
    def _(r):
        for b in range(_NBUF):
            k = r * _NBUF + b
            pltpu.async_copy(vbuf, acc.at[didx.at[pl.ds(k * _CHD, _CHD)]],
                             dsem, add=True)
        for b in range(_NBUF):
            k = r * _NBUF + b
            pltpu.make_async_copy(
                vbuf, acc.at[didx.at[pl.ds(k * _CHD, _CHD)]], dsem).wait()

    plsc.subcore_barrier()
    _copy_out(acc, out_hbm, c, s)


def _mm_body(x_ref, w_ref, o_ref):
    o_ref[...] = jnp.dot(x_ref[...], w_ref[...],
                         preferred_element_type=jnp.float32,
                         precision=lax.Precision.HIGHEST)


_mm = pl.pallas_call(
    _mm_body, out_shape=jax.ShapeDtypeStruct((_N, _D), jnp.float32))


def _prep_body(pdeg_ref, h_ref, dinv_ref, u_ref):
    deg = pdeg_ref[0:_N, 0:1] + pdeg_ref[_N:2 * _N, 0:1] + 1.0
    dinv = lax.rsqrt(deg)
    dinv_ref[...] = dinv
    u_ref[...] = h_ref[...] * dinv


_prep = pl.pallas_call(
    _prep_body,
    out_shape=(jax.ShapeDtypeStruct((_N, 1), jnp.float32),
               jax.ShapeDtypeStruct((_N, _D), jnp.float32)))


def _make_mid(dout):
    def body(s_ref, u_ref, dinv_ref, b_ref, g_ref, be_ref, w_ref, o_ref):
        dinv = dinv_ref[...]
        z = s_ref[0:_N, :] + s_ref[_N:2 * _N, :] + u_ref[...]
        h = dinv * z + b_ref[...]
        m = jnp.mean(h, axis=0, keepdims=True)
        ctr = h - m
        v = jnp.mean(ctr * ctr, axis=0, keepdims=True)
        hn = ctr * lax.rsqrt(v + 1e-5) * g_ref[...] + be_ref[...]
        hr = jnp.maximum(hn, 0.0)
        o_ref[...] = jnp.dot(hr, w_ref[...],
                             preferred_element_type=jnp.float32,
                             precision=lax.Precision.HIGHEST) * dinv

    return pl.pallas_call(
        body, out_shape=jax.ShapeDtypeStruct((_N, dout), jnp.float32))


_mid_d = _make_mid(_D)
_mid_p = _make_mid(_DOP)


def _final_body(s_ref, u_ref, dinv_ref, b_ref, o_ref):
    z = s_ref[0:_N, :] + s_ref[_N:2 * _N, :] + u_ref[...]
    h = dinv_ref[...] * z + b_ref[...]
    col = lax.broadcasted_iota(jnp.int32, (_N, _DOP), 1)
    valid = col < _DO
    hm = jnp.where(valid, h, jnp.float32(-1e30))
    mx = jnp.max(hm, axis=1, keepdims=True)
    ex = jnp.where(valid, jnp.exp(h - mx), 0.0)
    lse = jnp.log(jnp.sum(ex, axis=1, keepdims=True))
    o_ref[...] = (h - mx - lse)[:, 0:_DO]


_final = pl.pallas_call(
    _final_body, out_shape=jax.ShapeDtypeStruct((_N, _DO), jnp.float32))


def kernel(x, edge_index, W1, b1, g1, be1, W2, b2, g2, be2, W3, b3):
    src = edge_index[0]
    dst = edge_index[1]

    pdeg = _sc_deg(dst)
    h1 = _mm(x, W1)
    dinv, u1 = _prep(pdeg, h1)

    s1 = _sc_scatter_d(u1, src, dst)
    u2 = _mid_d(s1, u1, dinv, b1.reshape(1, -1), g1.reshape(1, -1),
                be1.reshape(1, -1), W2)

    s2 = _sc_scatter_d(u2, src, dst)
    w3p = jnp.concatenate(
        [W3, jnp.zeros((_D, _DOP - _DO), jnp.float32)], axis=1)
    u3 = _mid_p(s2, u2, dinv, b2.reshape(1, -1), g2.reshape(1, -1),
                be2.reshape(1, -1), w3p)

    s3 = _sc_scatter_p(u3, src, dst)
    b3p = jnp.concatenate(
        [b3, jnp.zeros((_DOP - _DO,), jnp.float32)]).reshape(1, -1)
    return _final(s3, u3, dinv, b3p)


# speedup vs baseline: 1.0336x; 1.0336x over previous
import functools

import jax
import jax.numpy as jnp
from jax import lax
from jax.experimental import pallas as pl
from jax.experimental.pallas import tpu as pltpu
from jax.experimental.pallas import tpu_sc as plsc

_N = 10000
_E = 320000
_D = 128
_DO = 40
_DOP = 128

_NC = 2
_NS = 16
_NW = _NC * _NS
_EPT = _E // _NW
_CH = 40
_NCHUNK = _EPT // _CH
_NBUF = 5
_K = 3
_ZR = 8
_CHD = 80
_NCHD = _EPT // _CHD
_RA = 624
_TAILB = _RA * _NS

_mesh = plsc.VectorSubcoreMesh(core_axis_name="c", subcore_axis_name="s")
_sc_params = pltpu.CompilerParams(needs_layout_passes=False)


def _zero_rows(zbuf, acc, s, feat):
    @pl.loop(0, _ZR)
    def _(i):
        @pl.loop(0, feat // 16)
        def _(j):
            zbuf[i, pl.ds(j * 16, 16)] = jnp.zeros((16,), jnp.float32)

    rbase = s * _RA

    @pl.loop(0, _RA // _ZR)
    def _(k):
        pltpu.sync_copy(zbuf, acc.at[pl.ds(rbase + k * _ZR, _ZR)])

    @pl.when(s == _NS - 1)
    def _():
        @pl.loop(0, (_N - _TAILB) // _ZR)
        def _(k):
            pltpu.sync_copy(zbuf, acc.at[pl.ds(_TAILB + k * _ZR, _ZR)])


def _copy_out(acc, out_hbm, c, s):
    rbase = s * _RA
    pltpu.sync_copy(acc.at[pl.ds(rbase, _RA)],
                    out_hbm.at[pl.ds(c * _N + rbase, _RA)])

    @pl.when(s == _NS - 1)
    def _():
        pltpu.sync_copy(acc.at[pl.ds(_TAILB, _N - _TAILB)],
                        out_hbm.at[pl.ds(c * _N + _TAILB, _N - _TAILB)])


def _make_sc_scatter(feat):
    @functools.partial(
        pl.kernel,
        out_type=jax.ShapeDtypeStruct((_NC * _N, feat), jnp.float32),
        mesh=_mesh,
        scratch_types=[
            pltpu.VMEM((_EPT,), jnp.int32),
            pltpu.VMEM((_EPT,), jnp.int32),
            pltpu.VMEM((_NBUF, _CH, feat), jnp.float32),
            pltpu.VMEM((_ZR, feat), jnp.float32),
            pltpu.VMEM_SHARED((_N, feat), jnp.float32),
            pltpu.SemaphoreType.DMA((_NBUF,)),
            pltpu.SemaphoreType.DMA((_NBUF,)),
        ],
        compiler_params=_sc_params,
    )
    def sc_scatter(u_hbm, src_hbm, dst_hbm, out_hbm, sidx, didx, rows, zbuf,
                   acc, gsem, ssem):
        c = lax.axis_index("c")
        s = lax.axis_index("s")
        wid = c * _NS + s
        pltpu.sync_copy(src_hbm.at[pl.ds(wid * _EPT, _EPT)], sidx)
        pltpu.sync_copy(dst_hbm.at[pl.ds(wid * _EPT, _EPT)], didx)
        _zero_rows(zbuf, acc, s, feat)
        plsc.subcore_barrier()

        def gather(g, b):
            pltpu.async_copy(u_hbm.at[sidx.at[pl.ds(g * _CH, _CH)]],
                             rows.at[b], gsem.at[b])

        def gather_wait(g, b):
            pltpu.make_async_copy(u_hbm.at[sidx.at[pl.ds(g * _CH, _CH)]],
                                  rows.at[b], gsem.at[b]).wait()

        for b in range(_NBUF):
            gather(b, b)

        @pl.loop(0, _NCHUNK // _NBUF)
        def _(r):
            g0 = r * _NBUF
            for b in range(_NBUF):
                g = g0 + b
                gather_wait(g, b)
                pltpu.sync_copy(rows.at[b],
                                acc.at[didx.at[pl.ds(g * _CH, _CH)]],
                                add=True)

                @pl.when(g + _NBUF < _NCHUNK)
                def _():
                    gather(g + _NBUF, b)

        plsc.subcore_barrier()
        _copy_out(acc, out_hbm, c, s)

    return sc_scatter


_sc_scatter_d = _make_sc_scatter(_D)
_sc_scatter_p = _sc_scatter_d


@functools.partial(
    pl.kernel,
    out_type=jax.ShapeDtypeStruct((_NC * _N, 16), jnp.float32),
    mesh=_mesh,
    scratch_types=[
        pltpu.VMEM((_EPT,), jnp.int32),
        pltpu.VMEM((_CHD, 16), jnp.float32),
        pltpu.VMEM((_ZR, 16), jnp.float32),
        pltpu.VMEM_SHARED((_N, 16), jnp.float32),
        pltpu.SemaphoreType.DMA,
    ],
    compiler_params=_sc_params,
)
def _sc_deg(dst_hbm, out_hbm, didx, vbuf, zbuf, acc, dsem):
    c = lax.axis_index("c")
    s = lax.axis_index("s")
    wid = c * _NS + s
    e0 = (lax.iota(jnp.int32, 16) == 0).astype(jnp.float32)

    @pl.loop(0, _CHD)
    def _(i):
        vbuf[i, pl.ds(0, 16)] = e0

    pltpu.sync_copy(dst_hbm.at[pl.ds(wid * _EPT, _EPT)], didx)
    _zero_rows(zbuf, acc, s, 16)
    plsc.subcore_barrier()

    @pl.loop(0, _NCHD // _NBUF)
    def _(r):
        for b in range(_NBUF):
            k = r * _NBUF + b
            pltpu.async_copy(vbuf, acc.at[didx.at[pl.ds(k * _CHD, _CHD)]],
                             dsem, add=True)
        for b in range(_NBUF):
            k = r * _NBUF + b
            pltpu.make_async_copy(
                vbuf, acc.at[didx.at[pl.ds(k * _CHD, _CHD)]], dsem).wait()

    plsc.subcore_barrier()
    _copy_out(acc, out_hbm, c, s)


def _mm_body(x_ref, w_ref, o_ref):
    o_ref[...] = jnp.dot(x_ref[...], w_ref[...],
                         preferred_element_type=jnp.float32,
                         precision=lax.Precision.HIGHEST)


_mm = pl.pallas_call(
    _mm_body, out_shape=jax.ShapeDtypeStruct((_N, _D), jnp.float32))


def _prep_body(pdeg_ref, h_ref, dinv_ref, u_ref):
    deg = pdeg_ref[0:_N, 0:1] + pdeg_ref[_N:2 * _N, 0:1] + 1.0
    dinv = lax.rsqrt(deg)
    dinv_ref[...] = dinv
    u_ref[...] = h_ref[...] * dinv


_prep = pl.pallas_call(
    _prep_body,
    out_shape=(jax.ShapeDtypeStruct((_N, 1), jnp.float32),
               jax.ShapeDtypeStruct((_N, _D), jnp.float32)))


def _make_mid(dout):
    def body(s_ref, u_ref, dinv_ref, b_ref, g_ref, be_ref, w_ref, o_ref):
        dinv = dinv_ref[...]
        z = s_ref[0:_N, :] + s_ref[_N:2 * _N, :] + u_ref[...]
        h = dinv * z + b_ref[...]
        m = jnp.mean(h, axis=0, keepdims=True)
        ctr = h - m
        v = jnp.mean(ctr * ctr, axis=0, keepdims=True)
        hn = ctr * lax.rsqrt(v + 1e-5) * g_ref[...] + be_ref[...]
        hr = jnp.maximum(hn, 0.0)
        o_ref[...] = jnp.dot(hr, w_ref[...],
                             preferred_element_type=jnp.float32,
                             precision=lax.Precision.HIGHEST) * dinv

    return pl.pallas_call(
        body, out_shape=jax.ShapeDtypeStruct((_N, dout), jnp.float32))


_mid_d = _make_mid(_D)
_mid_p = _make_mid(_DOP)


def _final_body(s_ref, u_ref, dinv_ref, b_ref, o_ref):
    z = s_ref[0:_N, :] + s_ref[_N:2 * _N, :] + u_ref[...]
    h = dinv_ref[...] * z + b_ref[...]
    col = lax.broadcasted_iota(jnp.int32, (_N, _DOP), 1)
    valid = col < _DO
    hm = jnp.where(valid, h, jnp.float32(-1e30))
    mx = jnp.max(hm, axis=1, keepdims=True)
    ex = jnp.where(valid, jnp.exp(h - mx), 0.0)
    lse = jnp.log(jnp.sum(ex, axis=1, keepdims=True))
    o_ref[...] = (h - mx - lse)[:, 0:_DO]


_final = pl.pallas_call(
    _final_body, out_shape=jax.ShapeDtypeStruct((_N, _DO), jnp.float32))


def kernel(x, edge_index, W1, b1, g1, be1, W2, b2, g2, be2, W3, b3):
    src = edge_index[0]
    dst = edge_index[1]

    pdeg = _sc_deg(dst)
    h1 = _mm(x, W1)
    dinv, u1 = _prep(pdeg, h1)

    s1 = _sc_scatter_d(u1, src, dst)
    u2 = _mid_d(s1, u1, dinv, b1.reshape(1, -1), g1.reshape(1, -1),
                be1.reshape(1, -1), W2)

    s2 = _sc_scatter_d(u2, src, dst)
    w3p = jnp.concatenate(
        [W3, jnp.zeros((_D, _DOP - _DO), jnp.float32)], axis=1)
    u3 = _mid_p(s2, u2, dinv, b2.reshape(1, -1), g2.reshape(1, -1),
                be2.reshape(1, -1), w3p)

    s3 = _sc_scatter_p(u3, src, dst)
    b3p = jnp.concatenate(
        [b3, jnp.zeros((_DOP - _DO,), jnp.float32)]).reshape(1, -1)
    return _final(s3, u3, dinv, b3p)
